# initial kernel scaffold (unmeasured)
import jax
import jax.numpy as jnp
from jax import lax
from jax.experimental import pallas as pl
from jax.experimental.pallas import tpu as pltpu

NDEV = 4


def kernel(O, Wo):
    B, S, H, D = O.shape
    K = H * D
    N = Wo.shape[1]
    S_out = S // NDEV

    O_b = O.reshape(B, S, K).astype(jnp.bfloat16)
    W_b = Wo.astype(jnp.bfloat16)
    P = jax.lax.dot_general(
        O_b, W_b,
        dimension_numbers=(((2,), (0,)), ((), ())),
        preferred_element_type=jnp.float32,
    ).astype(jnp.bfloat16)

    def body(p_ref, out_ref, acc_ref, recv_ref, va, vb, vo,
             send_sems, recv_sems, cp_sems):
        my = lax.axis_index("i")
        left = (my + NDEV - 1) % NDEV
        right = (my + 1) % NDEV

        barrier = pltpu.get_barrier_semaphore()
        for nbr in (left, right):
            pl.semaphore_signal(barrier, inc=1, device_id=(nbr,),
                                device_id_type=pl.DeviceIdType.MESH)
        pl.semaphore_wait(barrier, 2)

        def local_cp(src, dst, slot):
            cp = pltpu.make_async_copy(src, dst, cp_sems.at[slot])
            cp.start()
            return cp

        def ring_send(src, hop):
            rdma = pltpu.make_async_remote_copy(
                src_ref=src,
                dst_ref=recv_ref.at[hop],
                send_sem=send_sems.at[hop],
                recv_sem=recv_sems.at[hop],
                device_id=(right,),
                device_id_type=pl.DeviceIdType.MESH,
            )
            rdma.start()
            return rdma

        c0 = (my + NDEV - 1) % NDEV
        ring_send(p_ref.at[:, pl.ds(c0 * S_out, S_out), :], 0).wait()

        for h in (1, 2):
            c = (my + NDEV - 1 - h) % NDEV
            for b in range(B):
                cpa = local_cp(recv_ref.at[h - 1, b], va, 0)
                cpb = local_cp(p_ref.at[b, pl.ds(c * S_out, S_out), :], vb, 1)
                cpa.wait()
                cpb.wait()
                va[...] = va[...] + vb[...]
                local_cp(va, acc_ref.at[b], 0).wait()
            ring_send(acc_ref, h).wait()

        for b in range(B):
            cpa = local_cp(recv_ref.at[2, b], va, 0)
            cpb = local_cp(p_ref.at[b, pl.ds(my * S_out, S_out), :], vb, 1)
            cpa.wait()
            cpb.wait()
            vo[...] = va[...].astype(jnp.float32) + vb[...].astype(jnp.float32)
            local_cp(vo, out_ref.at[b], 0).wait()

    return pl.pallas_call(
        body,
        out_shape=jax.ShapeDtypeStruct((B, S_out, N), jnp.float32),
        in_specs=[pl.BlockSpec(memory_space=pl.ANY)],
        out_specs=pl.BlockSpec(memory_space=pl.ANY),
        scratch_shapes=[
            pltpu.MemorySpace.HBM((B, S_out, N), jnp.bfloat16),
            pltpu.MemorySpace.HBM((3, B, S_out, N), jnp.bfloat16),
            pltpu.VMEM((S_out, N), jnp.bfloat16),
            pltpu.VMEM((S_out, N), jnp.bfloat16),
            pltpu.VMEM((S_out, N), jnp.float32),
            pltpu.SemaphoreType.DMA((3,)),
            pltpu.SemaphoreType.DMA((3,)),
            pltpu.SemaphoreType.DMA((2,)),
        ],
        compiler_params=pltpu.CompilerParams(collective_id=0),
    )(P)


# baseline (device time: 1676163 ns/iter reference)
import jax
import jax.numpy as jnp
from jax import lax
from jax.experimental import pallas as pl
from jax.experimental.pallas import tpu as pltpu

NDEV = 4


def kernel(O, Wo):
    B, S, H, D = O.shape
    K = H * D
    N = Wo.shape[1]
    S_out = S // NDEV

    O_b = O.reshape(B, S, K).astype(jnp.bfloat16)
    W_b = Wo.astype(jnp.bfloat16)
    P = jax.lax.dot_general(
        O_b, W_b,
        dimension_numbers=(((2,), (0,)), ((), ())),
        preferred_element_type=jnp.float32,
    ).astype(jnp.bfloat16)

    def body(p_ref, out_ref, acc_ref, recv_ref, va, vb, vo,
             send_sems, recv_sems, cp_sems):
        del_unused = None
        my = lax.axis_index("i")
        left = (my + NDEV - 1) % NDEV
        right = (my + 1) % NDEV

        barrier = pltpu.get_barrier_semaphore()
        for nbr in (left, right):
            pl.semaphore_signal(barrier, inc=1, device_id=(nbr,),
                                device_id_type=pl.DeviceIdType.MESH)
        pl.semaphore_wait(barrier, 2)

        def local_cp(src, dst, slot):
            cp = pltpu.make_async_copy(src, dst, cp_sems.at[slot])
            cp.start()
            return cp

        def ring_send(src, hop):
            rdma = pltpu.make_async_remote_copy(
                src_ref=src,
                dst_ref=recv_ref.at[hop],
                send_sem=send_sems.at[hop],
                recv_sem=recv_sems.at[hop],
                device_id=(right,),
                device_id_type=pl.DeviceIdType.MESH,
            )
            rdma.start()
            return rdma

        c0 = (my + NDEV - 1) % NDEV
        ring_send(p_ref.at[:, pl.ds(c0 * S_out, S_out), :], 0).wait()

        for h in (1, 2):
            c = (my + NDEV - 1 - h) % NDEV
            for b in range(B):
                cpa = local_cp(recv_ref.at[h - 1, b], va, 0)
                cpb = local_cp(p_ref.at[b, pl.ds(c * S_out, S_out), :], vb, 1)
                cpa.wait()
                cpb.wait()
                va[...] = va[...] + vb[...]
                local_cp(va, acc_ref.at[b], 0).wait()
            ring_send(acc_ref, h).wait()

        for b in range(B):
            cpa = local_cp(recv_ref.at[2, b], va, 0)
            cpb = local_cp(p_ref.at[b, pl.ds(my * S_out, S_out), :], vb, 1)
            cpa.wait()
            cpb.wait()
            vo[...] = va[...].astype(jnp.float32) + vb[...].astype(jnp.float32)
            local_cp(vo, out_ref.at[b], 0).wait()

    out, _, _ = pl.pallas_call(
        body,
        out_shape=[
            jax.ShapeDtypeStruct((B, S_out, N), jnp.float32),
            jax.ShapeDtypeStruct((B, S_out, N), jnp.bfloat16),
            jax.ShapeDtypeStruct((3, B, S_out, N), jnp.bfloat16),
        ],
        in_specs=[pl.BlockSpec(memory_space=pl.ANY)],
        out_specs=[
            pl.BlockSpec(memory_space=pl.ANY),
            pl.BlockSpec(memory_space=pl.ANY),
            pl.BlockSpec(memory_space=pl.ANY),
        ],
        scratch_shapes=[
            pltpu.VMEM((S_out, N), jnp.bfloat16),
            pltpu.VMEM((S_out, N), jnp.bfloat16),
            pltpu.VMEM((S_out, N), jnp.float32),
            pltpu.SemaphoreType.DMA((3,)),
            pltpu.SemaphoreType.DMA((3,)),
            pltpu.SemaphoreType.DMA((2,)),
        ],
        compiler_params=pltpu.CompilerParams(collective_id=0),
    )(P)
    return out


# device time: 1284243 ns/iter; 1.3052x vs baseline; 1.3052x over previous
import jax
import jax.numpy as jnp
from jax import lax
from jax.experimental import pallas as pl
from jax.experimental.pallas import tpu as pltpu

NDEV = 4
NSUB_N = 2
NHOP = NDEV - 1


def kernel(O, Wo):
    B, S, H, D = O.shape
    K = H * D
    N = Wo.shape[1]
    S_out = S // NDEV
    NQ = N // NSUB_N
    NSUB = B * NSUB_N

    O3 = O.reshape(B, S, K).astype(jnp.bfloat16)
    W_b = Wo.astype(jnp.bfloat16)

    def body(o_ref, w_ref, out_ref, acc_ref, recv_ref,
             o_t, va, res, vo, ssems, rsems, csems):
        my = lax.axis_index("i")
        left = (my + NDEV - 1) % NDEV
        right = (my + 1) % NDEV

        barrier = pltpu.get_barrier_semaphore()
        for nbr in (left, right):
            pl.semaphore_signal(barrier, inc=1, device_id=(nbr,),
                                device_id_type=pl.DeviceIdType.MESH)
        pl.semaphore_wait(barrier, 2)

        def cp(src, dst, slot):
            c = pltpu.make_async_copy(src, dst, csems.at[slot])
            c.start()
            return c

        def rdma(h, sub):
            return pltpu.make_async_remote_copy(
                src_ref=acc_ref.at[h, sub],
                dst_ref=recv_ref.at[h, sub],
                send_sem=ssems.at[h, sub],
                recv_sem=rsems.at[h, sub],
                device_id=(right,),
                device_id_type=pl.DeviceIdType.MESH,
            )

        def load_o(b, c):
            return cp(o_ref.at[b, pl.ds(c * S_out, S_out), :], o_t, 0)

        def partial(nq):
            return jnp.dot(o_t[...], w_ref[:, nq * NQ:(nq + 1) * NQ],
                           preferred_element_type=jnp.float32)

        c0 = (my + NDEV - 1) % NDEV

        def hop0_b(b, _):
            load_o(b, c0).wait()
            for nq in range(NSUB_N):
                sub = b * NSUB_N + nq
                res[...] = partial(nq).astype(jnp.bfloat16)
                cp(res, acc_ref.at[0, sub], 1).wait()
                rdma(0, sub).start()
            return _

        lax.fori_loop(0, B, hop0_b, 0)

        for h in (1, 2):
            c = (my + NDEV - 1 - h) % NDEV

            def hop_b(b, _, h=h, c=c):
                load_o(b, c).wait()
                for nq in range(NSUB_N):
                    sub = b * NSUB_N + nq
                    rdma(h - 1, sub).wait_recv()
                    cp(recv_ref.at[h - 1, sub], va, 1).wait()
                    res[...] = (va[...].astype(jnp.float32)
                                + partial(nq)).astype(jnp.bfloat16)
                    cp(res, acc_ref.at[h, sub], 1).wait()
                    rdma(h, sub).start()
                return _

            lax.fori_loop(0, B, hop_b, 0)

        def final_b(b, _):
            load_o(b, my).wait()
            for nq in range(NSUB_N):
                sub = b * NSUB_N + nq
                rdma(2, sub).wait_recv()
                cp(recv_ref.at[2, sub], va, 1).wait()
                vo[...] = va[...].astype(jnp.float32) + partial(nq)
                cp(vo, out_ref.at[b, :, pl.ds(nq * NQ, NQ)], 1).wait()
            return _

        lax.fori_loop(0, B, final_b, 0)

        def drain(i, _):
            h = i // NSUB
            sub = i % NSUB
            rdma(h, sub).wait_send()
            return _

        lax.fori_loop(0, NHOP * NSUB, drain, 0)

    out, _, _ = pl.pallas_call(
        body,
        out_shape=[
            jax.ShapeDtypeStruct((B, S_out, N), jnp.float32),
            jax.ShapeDtypeStruct((NHOP, NSUB, S_out, NQ), jnp.bfloat16),
            jax.ShapeDtypeStruct((NHOP, NSUB, S_out, NQ), jnp.bfloat16),
        ],
        in_specs=[
            pl.BlockSpec(memory_space=pl.ANY),
            pl.BlockSpec(memory_space=pltpu.VMEM),
        ],
        out_specs=[
            pl.BlockSpec(memory_space=pl.ANY),
            pl.BlockSpec(memory_space=pl.ANY),
            pl.BlockSpec(memory_space=pl.ANY),
        ],
        scratch_shapes=[
            pltpu.VMEM((S_out, K), jnp.bfloat16),
            pltpu.VMEM((S_out, NQ), jnp.bfloat16),
            pltpu.VMEM((S_out, NQ), jnp.bfloat16),
            pltpu.VMEM((S_out, NQ), jnp.float32),
            pltpu.SemaphoreType.DMA((NHOP, NSUB)),
            pltpu.SemaphoreType.DMA((NHOP, NSUB)),
            pltpu.SemaphoreType.DMA((2,)),
        ],
        compiler_params=pltpu.CompilerParams(
            collective_id=0,
            vmem_limit_bytes=60 * 1024 * 1024,
        ),
    )(O3, W_b)
    return out
